# Initial kernel scaffold; baseline (speedup 1.0000x reference)
import jax
import jax.numpy as jnp
from jax.experimental import pallas as pl


def kernel(v, rbf0, e2, i, j, W_get_up, b_get_up, W_i, b_i, W_j, b_j, W_rbf1, W_rbf2, W_down, b_down, W_up, b_up, W_connect, b_connect, W_lin, b_lin, rb1_W1, rb1_b1, rb1_W2, rb1_b2, ra1_W1, ra1_b1, ra1_W2, ra1_b2, ra2_W1, ra2_b1, ra2_W2, ra2_b2, W_out):
    raise NotImplementedError("write your pallas kernel here")


# speedup vs baseline: 3.3691x; 3.3691x over previous
import functools

import jax
import jax.numpy as jnp
from jax import lax
from jax.experimental import pallas as pl
from jax.experimental.pallas import tpu as pltpu
from jax.experimental.pallas import tpu_sc as plsc

N = 10000
E = 320000
H = 128
NC = 2
NS = 16
NW = NC * NS
CHUNK = 128
NROWS = E // CHUNK
ROWS_PER_W = NROWS // NW
EXTRA = NROWS - ROWS_PER_W * NW
GROWS = ROWS_PER_W + 1
NPS = 624
NREM = N - NPS * NS


def _silu(x):
    return x / (1.0 + jnp.exp(-x))


def _linact_body(v_ref, w_ref, b_ref, o_ref):
    x = jnp.dot(v_ref[...], w_ref[...], preferred_element_type=jnp.float32)
    o_ref[...] = _silu(x + b_ref[...])


def _linact(v, W, b, br=1000):
    n = v.shape[0]
    return pl.pallas_call(
        _linact_body,
        grid=(n // br,),
        in_specs=[
            pl.BlockSpec((br, H), lambda bb: (bb, 0)),
            pl.BlockSpec((H, H), lambda bb: (0, 0)),
            pl.BlockSpec((1, H), lambda bb: (0, 0)),
        ],
        out_specs=pl.BlockSpec((br, H), lambda bb: (bb, 0)),
        out_shape=jax.ShapeDtypeStruct((n, H), jnp.float32),
    )(v, W, b.reshape(1, H))


def _edge_body(g_ref, r0_ref, w1_ref, w2_ref, wd_ref, bd_ref, o_ref):
    w1 = w1_ref[...]
    w2 = w2_ref[...]
    nr, be = w1.shape
    wc = w1[:, 0:1] * w2[0:1, :]
    for m in range(1, be):
        wc = wc + w1[:, m:m + 1] * w2[m:m + 1, :]
    r0 = r0_ref[...]
    rbf = r0[:, 0:1] * wc[0:1, :]
    for kk in range(1, nr):
        rbf = rbf + r0[:, kk:kk + 1] * wc[kk:kk + 1, :]
    x = g_ref[...] * rbf
    y = jnp.dot(x, wd_ref[...], preferred_element_type=jnp.float32)
    o_ref[...] = _silu(y + bd_ref[...])


def _edge(g, rbf0, W_rbf1, W_rbf2, W_down, b_down, beg=2000):
    nr = rbf0.shape[1]
    return pl.pallas_call(
        _edge_body,
        grid=(E // beg,),
        in_specs=[
            pl.BlockSpec((beg, H), lambda bb: (bb, 0)),
            pl.BlockSpec((beg, nr), lambda bb: (bb, 0)),
            pl.BlockSpec(W_rbf1.shape, lambda bb: (0, 0)),
            pl.BlockSpec(W_rbf2.shape, lambda bb: (0, 0)),
            pl.BlockSpec((H, H), lambda bb: (0, 0)),
            pl.BlockSpec((1, H), lambda bb: (0, 0)),
        ],
        out_specs=pl.BlockSpec((beg, H), lambda bb: (bb, 0)),
        out_shape=jax.ShapeDtypeStruct((E, H), jnp.float32),
    )(g, rbf0, W_rbf1, W_rbf2, W_down, b_down.reshape(1, H))


def _tail_body(sa_ref, sj_ref, v_ref,
               wgu_ref, bgu_ref, wi_ref, bi_ref, wup_ref, bup_ref,
               wcn_ref, bcn_ref, wln_ref, bln_ref,
               rb1w1_ref, rb1b1_ref, rb1w2_ref, rb1b2_ref,
               ra1w1_ref, ra1b1_ref, ra1w2_ref, ra1b2_ref,
               ra2w1_ref, ra2b1_ref, ra2w2_ref, ra2b2_ref,
               wout_ref, v2_ref, v1_ref):
    def mm(x, w_ref, b_ref):
        return jnp.dot(x, w_ref[...], preferred_element_type=jnp.float32) + b_ref[...]

    def res(x, w1_ref, b1_ref, w2_ref, b2_ref):
        return x + _silu(mm(_silu(mm(x, w1_ref, b1_ref)), w2_ref, b2_ref))

    v_old = v_ref[...]
    v_up = _silu(mm(sa_ref[0] + sa_ref[1], wgu_ref, bgu_ref))
    x_i = _silu(mm(v_old, wi_ref, bi_ref))
    xj = _silu(mm(sj_ref[0] + sj_ref[1], wup_ref, bup_ref))
    v2 = xj + x_i
    v2 = _silu(mm(v2, wcn_ref, bcn_ref)) + v_up
    v2 = res(v2, rb1w1_ref, rb1b1_ref, rb1w2_ref, rb1b2_ref)
    v2 = _silu(mm(v2, wln_ref, bln_ref)) + v_old
    v2 = res(v2, ra1w1_ref, ra1b1_ref, ra1w2_ref, ra1b2_ref)
    v2 = res(v2, ra2w1_ref, ra2b1_ref, ra2w2_ref, ra2b2_ref)
    v2_ref[...] = v2
    v1_ref[...] = jnp.dot(v2, wout_ref[...], preferred_element_type=jnp.float32)


def _tail(segA, segJ, v, W_get_up, b_get_up, W_i, b_i, W_up, b_up,
          W_connect, b_connect, W_lin, b_lin,
          rb1_W1, rb1_b1, rb1_W2, rb1_b2,
          ra1_W1, ra1_b1, ra1_W2, ra1_b2,
          ra2_W1, ra2_b1, ra2_W2, ra2_b2, W_out, br=1000):
    out_dim = W_out.shape[1]
    wspec = pl.BlockSpec((H, H), lambda bb: (0, 0))
    bspec = pl.BlockSpec((1, H), lambda bb: (0, 0))
    seg_spec = pl.BlockSpec((NC, br, H), lambda bb: (0, bb, 0))
    row_spec = pl.BlockSpec((br, H), lambda bb: (bb, 0))
    args = [segA, segJ, v,
            W_get_up, b_get_up, W_i, b_i, W_up, b_up,
            W_connect, b_connect, W_lin, b_lin,
            rb1_W1, rb1_b1, rb1_W2, rb1_b2,
            ra1_W1, ra1_b1, ra1_W2, ra1_b2,
            ra2_W1, ra2_b1, ra2_W2, ra2_b2, W_out]
    args = [a.reshape(1, H) if a.ndim == 1 else a for a in args]
    in_specs = [seg_spec, seg_spec, row_spec]
    for a in args[3:-1]:
        in_specs.append(wspec if a.shape == (H, H) else bspec)
    in_specs.append(pl.BlockSpec((H, out_dim), lambda bb: (0, 0)))
    return pl.pallas_call(
        _tail_body,
        grid=(N // br,),
        in_specs=in_specs,
        out_specs=[row_spec,
                   pl.BlockSpec((br, out_dim), lambda bb: (bb, 0))],
        out_shape=[jax.ShapeDtypeStruct((N, H), jnp.float32),
                   jax.ShapeDtypeStruct((N, out_dim), jnp.float32)],
    )(*args)


_MESH = dict(core_axis_name="c", subcore_axis_name="s")


def _sc_gather(table, j1):
    @functools.partial(
        pl.kernel,
        out_type=jax.ShapeDtypeStruct((E, H), jnp.float32),
        mesh=plsc.VectorSubcoreMesh(**_MESH),
        scratch_types=[
            pltpu.VMEM((GROWS * CHUNK,), jnp.int32),
            pltpu.VMEM((2, CHUNK, H), jnp.float32),
            pltpu.SemaphoreType.DMA,
        ],
    )
    def k(tab_hbm, j_hbm, out_hbm, idx_v, buf, osem):
        w = lax.axis_index("s") * NC + lax.axis_index("c")
        row0 = jnp.minimum(w * GROWS, NROWS - GROWS)
        e0 = pl.multiple_of(row0 * CHUNK, CHUNK)
        pltpu.sync_copy(j_hbm.at[pl.ds(e0, GROWS * CHUNK)], idx_v)

        def body(c, carry):
            slot = c % 2

            @pl.when(c >= 2)
            def _():
                pltpu.make_async_copy(
                    buf.at[0], out_hbm.at[pl.ds(0, CHUNK), :], osem).wait()

            coff = pl.multiple_of(c * CHUNK, CHUNK)
            pltpu.sync_copy(tab_hbm.at[idx_v.at[pl.ds(coff, CHUNK)]],
                            buf.at[slot])
            pltpu.async_copy(
                buf.at[slot],
                out_hbm.at[pl.ds(pl.multiple_of((row0 + c) * CHUNK, CHUNK),
                                 CHUNK), :],
                osem)
            return carry

        lax.fori_loop(0, GROWS, body, 0)
        pltpu.make_async_copy(buf.at[0], out_hbm.at[pl.ds(0, CHUNK), :], osem).wait()
        pltpu.make_async_copy(buf.at[0], out_hbm.at[pl.ds(0, CHUNK), :], osem).wait()

    return k(table, j1)


def _sc_segsum(vals, idx2, zrows):
    @functools.partial(
        pl.kernel,
        out_type=jax.ShapeDtypeStruct((NC, N, H), jnp.float32),
        mesh=plsc.VectorSubcoreMesh(**_MESH),
        scratch_types=[
            pltpu.VMEM((2, CHUNK), jnp.int32),
            pltpu.VMEM((2, CHUNK, H), jnp.float32),
            pltpu.VMEM_SHARED((N, H), jnp.float32),
            pltpu.SemaphoreType.DMA,
        ],
    )
    def k(vals_hbm, idx_hbm, z_hbm, out_hbm, idx_v, buf, shared, isem):
        c_ax = lax.axis_index("c")
        s_ax = lax.axis_index("s")
        w = s_ax * NC + c_ax
        rs = pl.multiple_of(s_ax * NPS, 8)
        pltpu.sync_copy(z_hbm.at[pl.ds(0, NPS), :], shared.at[pl.ds(rs, NPS), :])

        @pl.when(s_ax == 0)
        def _():
            pltpu.sync_copy(z_hbm.at[pl.ds(0, NREM), :],
                            shared.at[pl.ds(NPS * NS, NREM), :])

        plsc.subcore_barrier()

        nch = jnp.where(w < EXTRA, ROWS_PER_W + 1, ROWS_PER_W)
        row0 = jnp.where(w < EXTRA, (ROWS_PER_W + 1) * w, ROWS_PER_W * w + EXTRA)

        def fire(r, slot):
            e0 = pl.multiple_of(r * CHUNK, CHUNK)
            pltpu.async_copy(idx_hbm.at[pl.ds(e0, CHUNK)], idx_v.at[slot], isem)
            pltpu.async_copy(vals_hbm.at[pl.ds(e0, CHUNK), :], buf.at[slot], isem)

        fire(row0, 0)

        def body(c, carry):
            slot = c % 2
            pltpu.make_async_copy(
                idx_hbm.at[pl.ds(0, CHUNK)], idx_v.at[0], isem).wait()
            pltpu.make_async_copy(
                vals_hbm.at[pl.ds(0, CHUNK), :], buf.at[0], isem).wait()

            @pl.when(c + 1 < nch)
            def _():
                fire(row0 + c + 1, (c + 1) % 2)

            pltpu.sync_copy(buf.at[slot], shared.at[idx_v.at[slot]], add=True)
            return carry

        lax.fori_loop(0, nch, body, 0)
        plsc.subcore_barrier()
        pltpu.sync_copy(shared.at[pl.ds(rs, NPS), :],
                        out_hbm.at[c_ax, pl.ds(rs, NPS), :])

        @pl.when(s_ax == 0)
        def _():
            pltpu.sync_copy(shared.at[pl.ds(NPS * NS, NREM), :],
                            out_hbm.at[c_ax, pl.ds(NPS * NS, NREM), :])

    return k(vals, idx2, zrows)


def kernel(v, rbf0, e2, i, j, W_get_up, b_get_up, W_i, b_i, W_j, b_j,
           W_rbf1, W_rbf2, W_down, b_down, W_up, b_up, W_connect, b_connect,
           W_lin, b_lin, rb1_W1, rb1_b1, rb1_W2, rb1_b2,
           ra1_W1, ra1_b1, ra1_W2, ra1_b2, ra2_W1, ra2_b1, ra2_W2, ra2_b2,
           W_out):
    i1 = i.astype(jnp.int32)
    j1 = j.astype(jnp.int32)
    zrows = jnp.zeros((NPS, H), jnp.float32)

    x_j = _linact(v, W_j, b_j)
    g = _sc_gather(x_j, j1)
    y = _edge(g, rbf0, W_rbf1, W_rbf2, W_down, b_down)
    segJ = _sc_segsum(y, j1, zrows)
    segA = _sc_segsum(e2, i1, zrows)
    v2, v1 = _tail(segA, segJ, v, W_get_up, b_get_up, W_i, b_i, W_up, b_up,
                   W_connect, b_connect, W_lin, b_lin,
                   rb1_W1, rb1_b1, rb1_W2, rb1_b2,
                   ra1_W1, ra1_b1, ra1_W2, ra1_b2,
                   ra2_W1, ra2_b1, ra2_W2, ra2_b2, W_out)
    return (v2, v1)


# speedup vs baseline: 3.3749x; 1.0017x over previous
import functools

import jax
import jax.numpy as jnp
from jax import lax
from jax.experimental import pallas as pl
from jax.experimental.pallas import tpu as pltpu
from jax.experimental.pallas import tpu_sc as plsc

N = 10000
E = 320000
H = 128
NC = 2
NS = 16
NW = NC * NS
CHUNK = 128
NROWS = E // CHUNK
ROWS_PER_W = NROWS // NW
EXTRA = NROWS - ROWS_PER_W * NW
GROWS = ROWS_PER_W + 1
NPS = 624
NREM = N - NPS * NS


def _silu(x):
    return x / (1.0 + jnp.exp(-x))


def _linact_body(v_ref, w_ref, b_ref, o_ref):
    x = jnp.dot(v_ref[...], w_ref[...], preferred_element_type=jnp.float32)
    o_ref[...] = _silu(x + b_ref[...])


def _linact(v, W, b, br=1000):
    n = v.shape[0]
    return pl.pallas_call(
        _linact_body,
        grid=(n // br,),
        in_specs=[
            pl.BlockSpec((br, H), lambda bb: (bb, 0)),
            pl.BlockSpec((H, H), lambda bb: (0, 0)),
            pl.BlockSpec((1, H), lambda bb: (0, 0)),
        ],
        out_specs=pl.BlockSpec((br, H), lambda bb: (bb, 0)),
        out_shape=jax.ShapeDtypeStruct((n, H), jnp.float32),
    )(v, W, b.reshape(1, H))


def _edge_body(g_ref, r0_ref, w1_ref, w2_ref, wd_ref, bd_ref, o_ref):
    w1 = w1_ref[...]
    w2 = w2_ref[...]
    nr, be = w1.shape
    wc = w1[:, 0:1] * w2[0:1, :]
    for m in range(1, be):
        wc = wc + w1[:, m:m + 1] * w2[m:m + 1, :]
    r0 = r0_ref[...]
    rbf = r0[:, 0:1] * wc[0:1, :]
    for kk in range(1, nr):
        rbf = rbf + r0[:, kk:kk + 1] * wc[kk:kk + 1, :]
    x = g_ref[...] * rbf
    y = jnp.dot(x, wd_ref[...], preferred_element_type=jnp.float32)
    o_ref[...] = _silu(y + bd_ref[...])


def _edge(g, rbf0, W_rbf1, W_rbf2, W_down, b_down, beg=2000):
    nr = rbf0.shape[1]
    return pl.pallas_call(
        _edge_body,
        grid=(E // beg,),
        in_specs=[
            pl.BlockSpec((beg, H), lambda bb: (bb, 0)),
            pl.BlockSpec((beg, nr), lambda bb: (bb, 0)),
            pl.BlockSpec(W_rbf1.shape, lambda bb: (0, 0)),
            pl.BlockSpec(W_rbf2.shape, lambda bb: (0, 0)),
            pl.BlockSpec((H, H), lambda bb: (0, 0)),
            pl.BlockSpec((1, H), lambda bb: (0, 0)),
        ],
        out_specs=pl.BlockSpec((beg, H), lambda bb: (bb, 0)),
        out_shape=jax.ShapeDtypeStruct((E, H), jnp.float32),
    )(g, rbf0, W_rbf1, W_rbf2, W_down, b_down.reshape(1, H))


def _tail_body(sa_ref, sj_ref, v_ref,
               wgu_ref, bgu_ref, wi_ref, bi_ref, wup_ref, bup_ref,
               wcn_ref, bcn_ref, wln_ref, bln_ref,
               rb1w1_ref, rb1b1_ref, rb1w2_ref, rb1b2_ref,
               ra1w1_ref, ra1b1_ref, ra1w2_ref, ra1b2_ref,
               ra2w1_ref, ra2b1_ref, ra2w2_ref, ra2b2_ref,
               wout_ref, v2_ref, v1_ref):
    def mm(x, w_ref, b_ref):
        return jnp.dot(x, w_ref[...], preferred_element_type=jnp.float32) + b_ref[...]

    def res(x, w1_ref, b1_ref, w2_ref, b2_ref):
        return x + _silu(mm(_silu(mm(x, w1_ref, b1_ref)), w2_ref, b2_ref))

    v_old = v_ref[...]
    v_up = _silu(mm(sa_ref[0] + sa_ref[1], wgu_ref, bgu_ref))
    x_i = _silu(mm(v_old, wi_ref, bi_ref))
    xj = _silu(mm(sj_ref[0] + sj_ref[1], wup_ref, bup_ref))
    v2 = xj + x_i
    v2 = _silu(mm(v2, wcn_ref, bcn_ref)) + v_up
    v2 = res(v2, rb1w1_ref, rb1b1_ref, rb1w2_ref, rb1b2_ref)
    v2 = _silu(mm(v2, wln_ref, bln_ref)) + v_old
    v2 = res(v2, ra1w1_ref, ra1b1_ref, ra1w2_ref, ra1b2_ref)
    v2 = res(v2, ra2w1_ref, ra2b1_ref, ra2w2_ref, ra2b2_ref)
    v2_ref[...] = v2
    v1_ref[...] = jnp.dot(v2, wout_ref[...], preferred_element_type=jnp.float32)


def _tail(segA, segJ, v, W_get_up, b_get_up, W_i, b_i, W_up, b_up,
          W_connect, b_connect, W_lin, b_lin,
          rb1_W1, rb1_b1, rb1_W2, rb1_b2,
          ra1_W1, ra1_b1, ra1_W2, ra1_b2,
          ra2_W1, ra2_b1, ra2_W2, ra2_b2, W_out, br=1000):
    out_dim = W_out.shape[1]
    wspec = pl.BlockSpec((H, H), lambda bb: (0, 0))
    bspec = pl.BlockSpec((1, H), lambda bb: (0, 0))
    seg_spec = pl.BlockSpec((NC, br, H), lambda bb: (0, bb, 0))
    row_spec = pl.BlockSpec((br, H), lambda bb: (bb, 0))
    args = [segA, segJ, v,
            W_get_up, b_get_up, W_i, b_i, W_up, b_up,
            W_connect, b_connect, W_lin, b_lin,
            rb1_W1, rb1_b1, rb1_W2, rb1_b2,
            ra1_W1, ra1_b1, ra1_W2, ra1_b2,
            ra2_W1, ra2_b1, ra2_W2, ra2_b2, W_out]
    args = [a.reshape(1, H) if a.ndim == 1 else a for a in args]
    in_specs = [seg_spec, seg_spec, row_spec]
    for a in args[3:-1]:
        in_specs.append(wspec if a.shape == (H, H) else bspec)
    in_specs.append(pl.BlockSpec((H, out_dim), lambda bb: (0, 0)))
    return pl.pallas_call(
        _tail_body,
        grid=(N // br,),
        in_specs=in_specs,
        out_specs=[row_spec,
                   pl.BlockSpec((br, out_dim), lambda bb: (bb, 0))],
        out_shape=[jax.ShapeDtypeStruct((N, H), jnp.float32),
                   jax.ShapeDtypeStruct((N, out_dim), jnp.float32)],
    )(*args)


_MESH = dict(core_axis_name="c", subcore_axis_name="s")


def _sc_gather(table, j1):
    @functools.partial(
        pl.kernel,
        out_type=jax.ShapeDtypeStruct((E, H), jnp.float32),
        mesh=plsc.VectorSubcoreMesh(**_MESH),
        scratch_types=[
            pltpu.VMEM((GROWS * CHUNK,), jnp.int32),
            pltpu.VMEM((2, CHUNK, H), jnp.float32),
            pltpu.SemaphoreType.DMA,
        ],
    )
    def k(tab_hbm, j_hbm, out_hbm, idx_v, buf, osem):
        w = lax.axis_index("s") * NC + lax.axis_index("c")
        row0 = jnp.minimum(w * GROWS, NROWS - GROWS)
        e0 = pl.multiple_of(row0 * CHUNK, CHUNK)
        pltpu.sync_copy(j_hbm.at[pl.ds(e0, GROWS * CHUNK)], idx_v)

        def body(c, carry):
            slot = c % 2

            @pl.when(c >= 2)
            def _():
                pltpu.make_async_copy(
                    buf.at[0], out_hbm.at[pl.ds(0, CHUNK), :], osem).wait()

            coff = pl.multiple_of(c * CHUNK, CHUNK)
            pltpu.sync_copy(tab_hbm.at[idx_v.at[pl.ds(coff, CHUNK)]],
                            buf.at[slot])
            pltpu.async_copy(
                buf.at[slot],
                out_hbm.at[pl.ds(pl.multiple_of((row0 + c) * CHUNK, CHUNK),
                                 CHUNK), :],
                osem)
            return carry

        lax.fori_loop(0, GROWS, body, 0)
        pltpu.make_async_copy(buf.at[0], out_hbm.at[pl.ds(0, CHUNK), :], osem).wait()
        pltpu.make_async_copy(buf.at[0], out_hbm.at[pl.ds(0, CHUNK), :], osem).wait()

    return k(table, j1)


def _sc_segsum(vals, idx2, zrows):
    @functools.partial(
        pl.kernel,
        out_type=jax.ShapeDtypeStruct((NC, N, H), jnp.float32),
        mesh=plsc.VectorSubcoreMesh(**_MESH),
        scratch_types=[
            pltpu.VMEM((2, CHUNK), jnp.int32),
            pltpu.VMEM((2, CHUNK, H), jnp.float32),
            pltpu.VMEM_SHARED((N, H), jnp.float32),
            pltpu.SemaphoreType.DMA,
        ],
    )
    def k(vals_hbm, idx_hbm, z_hbm, out_hbm, idx_v, buf, shared, isem):
        c_ax = lax.axis_index("c")
        s_ax = lax.axis_index("s")
        w = s_ax * NC + c_ax
        rs = pl.multiple_of(s_ax * NPS, 8)
        pltpu.sync_copy(z_hbm.at[pl.ds(0, NPS), :], shared.at[pl.ds(rs, NPS), :])

        @pl.when(s_ax == 0)
        def _():
            pltpu.sync_copy(z_hbm.at[pl.ds(0, NREM), :],
                            shared.at[pl.ds(NPS * NS, NREM), :])

        plsc.subcore_barrier()

        nch = jnp.where(w < EXTRA, ROWS_PER_W + 1, ROWS_PER_W)
        row0 = jnp.where(w < EXTRA, (ROWS_PER_W + 1) * w, ROWS_PER_W * w + EXTRA)

        def fire(r, slot):
            e0 = pl.multiple_of(r * CHUNK, CHUNK)
            pltpu.async_copy(idx_hbm.at[pl.ds(e0, CHUNK)], idx_v.at[slot], isem)
            pltpu.async_copy(vals_hbm.at[pl.ds(e0, CHUNK), :], buf.at[slot], isem)

        fire(row0, 0)

        def body(c, carry):
            slot = c % 2
            pltpu.make_async_copy(
                idx_hbm.at[pl.ds(0, CHUNK)], idx_v.at[0], isem).wait()
            pltpu.make_async_copy(
                vals_hbm.at[pl.ds(0, CHUNK), :], buf.at[0], isem).wait()

            @pl.when(c + 1 < nch)
            def _():
                fire(row0 + c + 1, (c + 1) % 2)

            pltpu.sync_copy(buf.at[slot], shared.at[idx_v.at[slot]], add=True)
            return carry

        lax.fori_loop(0, nch, body, 0)
        plsc.subcore_barrier()
        pltpu.sync_copy(shared.at[pl.ds(rs, NPS), :],
                        out_hbm.at[c_ax, pl.ds(rs, NPS), :])

        @pl.when(s_ax == 0)
        def _():
            pltpu.sync_copy(shared.at[pl.ds(NPS * NS, NREM), :],
                            out_hbm.at[c_ax, pl.ds(NPS * NS, NREM), :])

    return k(vals, idx2, zrows)


def kernel(v, rbf0, e2, i, j, W_get_up, b_get_up, W_i, b_i, W_j, b_j,
           W_rbf1, W_rbf2, W_down, b_down, W_up, b_up, W_connect, b_connect,
           W_lin, b_lin, rb1_W1, rb1_b1, rb1_W2, rb1_b2,
           ra1_W1, ra1_b1, ra1_W2, ra1_b2, ra2_W1, ra2_b1, ra2_W2, ra2_b2,
           W_out):
    i1 = i.astype(jnp.int32)
    j1 = j.astype(jnp.int32)
    zrows = jnp.zeros((NPS, H), jnp.float32)

    segA = _sc_segsum(e2, i1, zrows)
    x_j = _linact(v, W_j, b_j)
    g = _sc_gather(x_j, j1)
    y = _edge(g, rbf0, W_rbf1, W_rbf2, W_down, b_down)
    segJ = _sc_segsum(y, j1, zrows)
    v2, v1 = _tail(segA, segJ, v, W_get_up, b_get_up, W_i, b_i, W_up, b_up,
                   W_connect, b_connect, W_lin, b_lin,
                   rb1_W1, rb1_b1, rb1_W2, rb1_b2,
                   ra1_W1, ra1_b1, ra1_W2, ra1_b2,
                   ra2_W1, ra2_b1, ra2_W2, ra2_b2, W_out)
    return (v2, v1)
